# Initial kernel scaffold; baseline (speedup 1.0000x reference)
import jax
import jax.numpy as jnp
from jax.experimental import pallas as pl


def kernel(batch_pred, batch_group):
    raise NotImplementedError("write your pallas kernel here")


# speedup vs baseline: 370.2380x; 370.2380x over previous
import jax
import jax.numpy as jnp
from jax import lax
from jax.experimental import pallas as pl
from jax.experimental.pallas import tpu as pltpu

NG = 4
R = 128
C = 128
N = R * C


def _wd_kernel(pred_ref, grp_ref, out_ref):
    v = pred_ref[...]
    g = grp_ref[...].astype(jnp.float32)

    row = lax.broadcasted_iota(jnp.int32, (R, C), 0)
    col = lax.broadcasted_iota(jnp.int32, (R, C), 1)

    ii = lax.broadcasted_iota(jnp.int32, (R, R), 0)
    jj = lax.broadcasted_iota(jnp.int32, (R, R), 1)
    upper_incl = (ii <= jj).astype(jnp.float32)
    lower_strict = (jj < ii).astype(jnp.float32)

    def cumsum2d(x):
        rowc = jnp.dot(x, upper_incl, preferred_element_type=jnp.float32)
        totals = rowc[:, C - 1:C]
        prefix = jnp.dot(lower_strict, totals,
                         preferred_element_type=jnp.float32)
        return rowc + prefix

    masks = [(g == float(gi)).astype(jnp.float32) for gi in range(NG)]
    cums = [cumsum2d(m) for m in masks]
    counts = [jnp.sum(m) for m in masks]
    max_len = counts[0]
    for gi in range(1, NG):
        max_len = jnp.maximum(max_len, counts[gi])

    weight = jnp.zeros((R, C), jnp.float32)
    for gi in range(NG):
        n = jnp.maximum(counts[gi], 1.0)
        q = jnp.floor(max_len / n)
        q = jnp.where(q * n > max_len, q - 1.0, q)
        q = jnp.where((q + 1.0) * n <= max_len, q + 1.0, q)
        r_extra = max_len - q * n
        w_g = q + (cums[gi] - 1.0 < r_extra).astype(jnp.float32)
        weight = weight + masks[gi] * w_g

    payload = g * 65536.0 + weight

    def partner_of(x, bit_set, dist, axis):
        size = (R, C)[axis]
        fwd = pltpu.roll(x, size - dist, axis)
        bwd = pltpu.roll(x, dist, axis)
        return jnp.where(bit_set, bwd, fwd)

    for K in range(1, 15):
        if K < 7:
            dirbit = (col >> K) & 1
        elif K < 14:
            dirbit = (row >> (K - 7)) & 1
        else:
            dirbit = jnp.zeros((R, C), jnp.int32)
        up = dirbit == 0
        for j in range(K - 1, -1, -1):
            if j < 7:
                bit = ((col >> j) & 1) == 1
                axis, dist = 1, 1 << j
            else:
                bit = ((row >> (j - 7)) & 1) == 1
                axis, dist = 0, 1 << (j - 7)
            pv = partner_of(v, bit, dist, axis)
            pw = partner_of(payload, bit, dist, axis)
            keep_min = up ^ bit
            take = (keep_min & (pv < v)) | (~keep_min & (pv > v))
            v = jnp.where(take, pv, v)
            payload = jnp.where(take, pw, payload)

    g_sorted = jnp.floor(payload * (1.0 / 65536.0))
    w_sorted = payload - g_sorted * 65536.0
    cw = [cumsum2d(w_sorted * (g_sorted == float(gi)).astype(jnp.float32))
          for gi in range(NG)]

    s_abs = jnp.zeros((R, C), jnp.float32)
    for i in range(NG - 1):
        for j in range(i + 1, NG):
            s_abs = s_abs + jnp.abs(cw[i] - cw[j])

    nxt_lane = pltpu.roll(v, C - 1, 1)
    nxt_row = pltpu.roll(v, R - 1, 0)
    v_next = jnp.where(col == C - 1, nxt_row[:, 0:1], nxt_lane)
    dv = v_next - v
    total = jnp.sum(dv * s_abs)

    npairs = NG * (NG - 1) // 2
    out_ref[...] = jnp.broadcast_to(total / (float(npairs) * max_len), (1, 1))


def kernel(batch_pred, batch_group):
    v2 = batch_pred.reshape(R, C)
    g2 = batch_group.reshape(R, C)
    out = pl.pallas_call(
        _wd_kernel,
        out_shape=jax.ShapeDtypeStruct((1, 1), jnp.float32),
        in_specs=[
            pl.BlockSpec(memory_space=pltpu.VMEM),
            pl.BlockSpec(memory_space=pltpu.VMEM),
        ],
        out_specs=pl.BlockSpec(memory_space=pltpu.VMEM),
    )(v2, g2)
    return out[0, 0]
